# baseline (speedup 1.0000x reference)
import functools

import jax
import jax.numpy as jnp
from jax import lax
from jax.experimental import pallas as pl
from jax.experimental.pallas import tpu as pltpu
from jax.experimental.pallas import tpu_sc as plsc

D = 32
NC, NS = 2, 16
NW = NC * NS
CHUNK = 128
K = 2
WPR = 128 // D


def _gather_rows(idx, tablew, b_per_w, n_chunks):
    n_outer = n_chunks // K
    rows_per_step = K * CHUNK
    wide_per_step = rows_per_step // WPR
    mesh = plsc.VectorSubcoreMesh(core_axis_name="c", subcore_axis_name="s")

    @functools.partial(
        pl.kernel,
        out_type=jax.ShapeDtypeStruct((NW * b_per_w // WPR, 128), jnp.float32),
        mesh=mesh,
        compiler_params=pltpu.CompilerParams(use_tc_tiling_on_sc=True),
        scratch_types=[
            pltpu.VMEM((n_chunks, CHUNK), jnp.int32),
            pltpu.VMEM((rows_per_step,), jnp.int32),
            pltpu.VMEM((rows_per_step,), jnp.int32),
            pltpu.VMEM((rows_per_step, 128), jnp.float32),
            pltpu.VMEM((rows_per_step, 128), jnp.float32),
            pltpu.VMEM((wide_per_step, 128), jnp.float32),
            pltpu.VMEM((wide_per_step, 128), jnp.float32),
            pltpu.SemaphoreType.DMA,
            pltpu.SemaphoreType.DMA,
            pltpu.SemaphoreType.DMA,
            pltpu.SemaphoreType.DMA,
        ],
    )
    def k(idx_hbm, table_hbm, out_hbm, idx_v, w0, w1, g0, g1, o0, o1,
          gs0, gs1, os0, os1):
        wid = lax.axis_index("s") * NC + lax.axis_index("c")
        wbase = wid * (b_per_w // WPR)
        pltpu.sync_copy(idx_hbm.at[wid], idx_v)
        widx = (w0, w1)
        gbuf = (g0, g1)
        obuf = (o0, o1)
        gsem = (gs0, gs1)
        osem = (os0, os1)

        def build_widx(j, b):
            for kk in range(K):
                for m in range(CHUNK // 16):
                    v = idx_v[j * K + kk, pl.ds(m * 16, 16)]
                    widx[b][pl.ds(kk * CHUNK + m * 16, 16)] = (
                        lax.shift_right_logical(v, 2))

        def fire(j, b):
            for kk in range(K):
                pltpu.async_copy(
                    table_hbm.at[widx[b].at[pl.ds(kk * CHUNK, CHUNK)]],
                    gbuf[b].at[pl.ds(kk * CHUNK, CHUNK)],
                    gsem[b],
                )

        def drain_gathers(j, b):
            for kk in range(K):
                pltpu.make_async_copy(
                    table_hbm.at[widx[b].at[pl.ds(kk * CHUNK, CHUNK)]],
                    gbuf[b].at[pl.ds(kk * CHUNK, CHUNK)],
                    gsem[b],
                ).wait()

        def select_repack(j, b):
            g, o = gbuf[b], obuf[b]
            for kk in range(K):
                for m in range(CHUNK // 16):
                    qv = lax.bitwise_and(
                        idx_v[j * K + kk, pl.ds(m * 16, 16)], 3) * D
                    for l in range(16):
                        lane = m * 16 + l
                        q32 = qv[l]
                        for h in range(D // 16):
                            o[kk * 32 + lane // WPR,
                              pl.ds((lane % WPR) * D + h * 16, 16)] = (
                                g[kk * CHUNK + lane,
                                  pl.ds(q32 + h * 16, 16)])

        def out_copy(j, b):
            pltpu.async_copy(
                obuf[b],
                out_hbm.at[pl.ds(wbase + j * wide_per_step, wide_per_step)],
                osem[b],
            )

        def drain_out(j, b):
            pltpu.make_async_copy(
                obuf[b],
                out_hbm.at[pl.ds(wbase + j * wide_per_step, wide_per_step)],
                osem[b],
            ).wait()

        build_widx(0, 0)
        fire(0, 0)

        def body(j, _):
            par = lax.rem(j, 2)
            for b in (0, 1):
                @pl.when(par == b)
                def _():
                    @pl.when(j >= 2)
                    def _():
                        drain_out(j - 2, b)
                    build_widx(j, b)
                    fire(j, b)
                    drain_gathers(j - 1, 1 - b)
                    select_repack(j - 1, 1 - b)
                    out_copy(j - 1, 1 - b)
            return 0

        lax.fori_loop(1, n_outer, body, 0, unroll=False)
        last = n_outer - 1
        lastbuf = last % 2
        if n_outer >= 2:
            drain_out(last - 1, 1 - lastbuf)
        drain_gathers(last, lastbuf)
        select_repack(last, lastbuf)
        out_copy(last, lastbuf)
        drain_out(last, lastbuf)

    return k(idx, tablew)


def kernel(indices, table):
    B, S = indices.shape
    total = B * S
    b_per_w = total // NW
    n_chunks = b_per_w // CHUNK
    idx = indices.astype(jnp.int32).reshape(NW, n_chunks, CHUNK)
    tablew = table.reshape(table.shape[0] // WPR, 128)
    out = _gather_rows(idx, tablew, b_per_w, n_chunks)
    return out.reshape(B, S, D)


# speedup vs baseline: 1.2448x; 1.2448x over previous
import functools

import jax
import jax.numpy as jnp
from jax import lax
from jax.experimental import pallas as pl
from jax.experimental.pallas import tpu as pltpu
from jax.experimental.pallas import tpu_sc as plsc

D = 32
NC, NS = 2, 16
NW = NC * NS
CHUNK = 256
K = 4


def _gather_rows(idx, table, b_per_w, n_chunks):
    n_outer = n_chunks // K
    rows_per_step = K * CHUNK
    mesh = plsc.VectorSubcoreMesh(core_axis_name="c", subcore_axis_name="s")

    @functools.partial(
        pl.kernel,
        out_type=jax.ShapeDtypeStruct((NW * b_per_w, D), jnp.float32),
        mesh=mesh,
        compiler_params=pltpu.CompilerParams(use_tc_tiling_on_sc=False),
        scratch_types=[
            pltpu.VMEM((n_chunks, CHUNK), jnp.int32),
            pltpu.VMEM((rows_per_step, D), jnp.float32),
            pltpu.VMEM((rows_per_step, D), jnp.float32),
            pltpu.SemaphoreType.DMA,
            pltpu.SemaphoreType.DMA,
            pltpu.SemaphoreType.DMA,
            pltpu.SemaphoreType.DMA,
        ],
    )
    def k(idx_hbm, table_hbm, out_hbm, idx_v, rows0, rows1, g0, g1, o0, o1):
        wid = lax.axis_index("s") * NC + lax.axis_index("c")
        base = wid * b_per_w
        pltpu.sync_copy(idx_hbm.at[wid], idx_v)
        rows = (rows0, rows1)
        gsem = (g0, g1)
        osem = (o0, o1)

        def fire(j, buf):
            for kk in range(K):
                pltpu.async_copy(
                    table_hbm.at[idx_v.at[j * K + kk]],
                    rows[buf].at[pl.ds(kk * CHUNK, CHUNK)],
                    gsem[buf],
                )

        def drain_gathers(j, buf):
            for kk in range(K):
                pltpu.make_async_copy(
                    table_hbm.at[idx_v.at[j * K + kk]],
                    rows[buf].at[pl.ds(kk * CHUNK, CHUNK)],
                    gsem[buf],
                ).wait()

        def out_copy(j, buf):
            pltpu.async_copy(
                rows[buf],
                out_hbm.at[pl.ds(base + j * rows_per_step, rows_per_step)],
                osem[buf],
            )

        def drain_out(j, buf):
            pltpu.make_async_copy(
                rows[buf],
                out_hbm.at[pl.ds(base + j * rows_per_step, rows_per_step)],
                osem[buf],
            ).wait()

        fire(0, 0)

        def body(j, _):
            buf = lax.rem(j, 2)
            for b in (0, 1):
                @pl.when(buf == b)
                def _():
                    @pl.when(j >= 2)
                    def _():
                        drain_out(j - 2, b)
                    fire(j, b)
                    drain_gathers(j - 1, 1 - b)
                    out_copy(j - 1, 1 - b)
            return 0

        lax.fori_loop(1, n_outer, body, 0, unroll=False)
        last = n_outer - 1
        lastbuf = last % 2
        if n_outer >= 2:
            drain_out(last - 1, 1 - lastbuf)
        drain_gathers(last, lastbuf)
        out_copy(last, lastbuf)
        drain_out(last, lastbuf)

    return k(idx, table)


def kernel(indices, table):
    B, S = indices.shape
    total = B * S
    b_per_w = total // NW
    n_chunks = b_per_w // CHUNK
    idx = indices.astype(jnp.int32).reshape(NW, n_chunks, CHUNK)
    out = _gather_rows(idx, table, b_per_w, n_chunks)
    return out.reshape(B, S, D)
